# Initial kernel scaffold; baseline (speedup 1.0000x reference)
import jax
import jax.numpy as jnp
from jax.experimental import pallas as pl


def kernel(X, adj, W_theta_w, W_theta_b):
    raise NotImplementedError("write your pallas kernel here")


# speedup vs baseline: 10.0225x; 10.0225x over previous
import jax
import jax.numpy as jnp
from jax import lax
from jax.experimental import pallas as pl

_N = 10000
_DH = 16
_TOPK = 5
_ALPHA = 0.5
_R = 40


def _xtheta_body(x_ref, w_ref, b_ref, o_ref):
    z = lax.dot_general(
        x_ref[...], w_ref[...], (((1,), (1,)), ((), ())),
        preferred_element_type=jnp.float32)
    o_ref[...] = jnp.maximum(z + b_ref[...], 0.0)


def _fused_body(xth_ref, adj_ref, p_ref, a_ref):
    r = pl.program_id(0)
    xr = xth_ref[pl.ds(r * _R, _R), :]
    z = lax.dot_general(
        xr, xth_ref[...], (((1,), (1,)), ((), ())),
        preferred_element_type=jnp.float32)
    a = jnp.tanh(z)
    a_ref[...] = a

    cols = lax.broadcasted_iota(jnp.int32, (_R, _N), 1)
    work = a
    keep = jnp.zeros((_R, _N), dtype=jnp.bool_)
    for _ in range(_TOPK):
        m = jnp.max(work, axis=1, keepdims=True)
        sel = jnp.min(
            jnp.where(work == m, cols, jnp.int32(2 ** 30)),
            axis=1, keepdims=True)
        hit = cols == sel
        keep = jnp.logical_or(keep, hit)
        work = jnp.where(hit, jnp.float32(-2.0), work)

    p_ref[...] = adj_ref[...] + jnp.where(keep, _ALPHA * a, jnp.float32(0.0))


@jax.jit
def kernel(X, adj, W_theta_w, W_theta_b):
    xth = pl.pallas_call(
        _xtheta_body,
        out_shape=jax.ShapeDtypeStruct((_N, _DH), jnp.float32),
    )(X, W_theta_w, W_theta_b.reshape(1, _DH))

    P, A = pl.pallas_call(
        _fused_body,
        grid=(_N // _R,),
        in_specs=[
            pl.BlockSpec((_N, _DH), lambda r: (0, 0)),
            pl.BlockSpec((_R, _N), lambda r: (r, 0)),
        ],
        out_specs=[
            pl.BlockSpec((_R, _N), lambda r: (r, 0)),
            pl.BlockSpec((_R, _N), lambda r: (r, 0)),
        ],
        out_shape=[
            jax.ShapeDtypeStruct((_N, _N), jnp.float32),
            jax.ShapeDtypeStruct((_N, _N), jnp.float32),
        ],
    )(xth, adj)
    return P, A
